# Initial kernel scaffold; baseline (speedup 1.0000x reference)
import jax
import jax.numpy as jnp
from jax.experimental import pallas as pl


def kernel(hash_indices, table_0, table_1, table_2, table_3, table_4, table_5, table_6, table_7):
    raise NotImplementedError("write your pallas kernel here")


# speedup vs baseline: 5.8479x; 5.8479x over previous
import functools

import jax
import jax.numpy as jnp
from jax import lax
from jax.experimental import pallas as pl
from jax.experimental.pallas import tpu as pltpu
from jax.experimental.pallas import tpu_sc as plsc

_TABLE_SIZES = (100003, 100019, 100043, 100057, 100069, 100103, 100109,
                100129)
_NUM_HEADS = 8
_EMBED_DIM = 32
_NC = 2
_NS = 16
_NW = _NC * _NS
_STREAM = 128
_NSTREAM = 8
_CHUNK = _STREAM * _NSTREAM


@functools.lru_cache(maxsize=None)
def _build(R: int):
    assert R % (_NW * _CHUNK) == 0
    chunks_per_worker = R // (_NW * _CHUNK)
    n_stream_rows = R // _STREAM
    mesh = plsc.VectorSubcoreMesh(core_axis_name="c", subcore_axis_name="s")

    @functools.partial(
        pl.kernel,
        out_type=jax.ShapeDtypeStruct((n_stream_rows, _STREAM, _EMBED_DIM),
                                      jnp.float32),
        mesh=mesh,
        compiler_params=pltpu.CompilerParams(use_tc_tiling_on_sc=False),
        scratch_types=[
            pltpu.VMEM((_NSTREAM, _STREAM), jnp.int32),
            pltpu.VMEM((_NSTREAM, _STREAM, _EMBED_DIM), jnp.float32),
            pltpu.SemaphoreType.DMA,
        ],
    )
    def gather_kernel(table_hbm, idx_hbm, out_hbm, idx_v, rows_v, sem):
        wid = lax.axis_index("s") * _NC + lax.axis_index("c")

        def step(ci, carry):
            sbase = (wid * chunks_per_worker + ci) * _NSTREAM
            pltpu.sync_copy(idx_hbm.at[pl.ds(sbase, _NSTREAM)], idx_v)
            copies = [
                pltpu.async_copy(table_hbm.at[idx_v.at[k]], rows_v.at[k],
                                 sem)
                for k in range(_NSTREAM)
            ]
            for c in copies:
                c.wait()
            pltpu.sync_copy(rows_v, out_hbm.at[pl.ds(sbase, _NSTREAM)])
            return carry

        lax.fori_loop(0, chunks_per_worker, step, 0)

    return gather_kernel


def kernel(hash_indices, table_0, table_1, table_2, table_3, table_4,
           table_5, table_6, table_7):
    B, S, H = hash_indices.shape
    T = B * S
    tables = [table_0, table_1, table_2, table_3, table_4, table_5, table_6,
              table_7]
    offs = jnp.array(
        [sum(_TABLE_SIZES[:h]) for h in range(_NUM_HEADS)], dtype=jnp.int32)
    idx_flat = (hash_indices.reshape(T, H) + offs[None, :]).reshape(
        T * H // _STREAM, _STREAM)
    table_cat = jnp.concatenate(tables, axis=0)
    out = _build(T * H)(table_cat, idx_flat)
    return out.reshape(B, S, _NUM_HEADS * _EMBED_DIM)


# speedup vs baseline: 6.0080x; 1.0274x over previous
import functools

import jax
import jax.numpy as jnp
from jax import lax
from jax.experimental import pallas as pl
from jax.experimental.pallas import tpu as pltpu
from jax.experimental.pallas import tpu_sc as plsc

_TABLE_SIZES = (100003, 100019, 100043, 100057, 100069, 100103, 100109,
                100129)
_NUM_HEADS = 8
_EMBED_DIM = 32
_NC = 2
_NS = 16
_NW = _NC * _NS
_STREAM = 128
_NSTREAM = 8
_CHUNK = _STREAM * _NSTREAM


@functools.lru_cache(maxsize=None)
def _build(R: int):
    assert R % (_NW * _CHUNK) == 0
    cpw = R // (_NW * _CHUNK)
    assert cpw % 2 == 0
    n_stream_rows = R // _STREAM
    mesh = plsc.VectorSubcoreMesh(core_axis_name="c", subcore_axis_name="s")

    @functools.partial(
        pl.kernel,
        out_type=jax.ShapeDtypeStruct((n_stream_rows, _STREAM, _EMBED_DIM),
                                      jnp.float32),
        mesh=mesh,
        compiler_params=pltpu.CompilerParams(use_tc_tiling_on_sc=False),
        scratch_types=[
            pltpu.VMEM((2, _NSTREAM, _STREAM), jnp.int32),
            pltpu.VMEM((2, _NSTREAM, _STREAM, _EMBED_DIM), jnp.float32),
            pltpu.SemaphoreType.DMA((2,)),
            pltpu.SemaphoreType.DMA((2,)),
        ],
    )
    def gather_kernel(table_hbm, idx_hbm, out_hbm, idx_v, rows_v, gsem,
                      osem):
        wid = lax.axis_index("s") * _NC + lax.axis_index("c")
        w0 = wid * cpw

        def fire(g, b):
            sbase = (w0 + g) * _NSTREAM
            pltpu.sync_copy(idx_hbm.at[pl.ds(sbase, _NSTREAM)], idx_v.at[b])
            for k_ in range(_NSTREAM):
                pltpu.async_copy(table_hbm.at[idx_v.at[b, k_]],
                                 rows_v.at[b, k_], gsem.at[b])

        def drain_gather(b):
            for k_ in range(_NSTREAM):
                pltpu.make_async_copy(table_hbm.at[idx_v.at[b, k_]],
                                      rows_v.at[b, k_], gsem.at[b]).wait()

        def write_out(g, b):
            sbase = (w0 + g) * _NSTREAM
            pltpu.async_copy(rows_v.at[b],
                             out_hbm.at[pl.ds(sbase, _NSTREAM)], osem.at[b])

        def wait_out(g, b):
            sbase = (w0 + g) * _NSTREAM
            pltpu.make_async_copy(rows_v.at[b],
                                  out_hbm.at[pl.ds(sbase, _NSTREAM)],
                                  osem.at[b]).wait()

        fire(0, 0)

        def body(j2, carry):
            for b in range(2):
                g = j2 * 2 + b
                nb = 1 - b

                @pl.when(g + 1 < cpw)
                def _():
                    @pl.when(g >= 1)
                    def _():
                        wait_out(g - 1, nb)
                    fire(g + 1, nb)

                drain_gather(b)
                write_out(g, b)
            return carry

        lax.fori_loop(0, cpw // 2, body, 0)
        wait_out(cpw - 2, 0)
        wait_out(cpw - 1, 1)

    return gather_kernel


def kernel(hash_indices, table_0, table_1, table_2, table_3, table_4,
           table_5, table_6, table_7):
    B, S, H = hash_indices.shape
    T = B * S
    tables = [table_0, table_1, table_2, table_3, table_4, table_5, table_6,
              table_7]
    offs = jnp.array(
        [sum(_TABLE_SIZES[:h]) for h in range(_NUM_HEADS)], dtype=jnp.int32)
    idx_flat = (hash_indices.reshape(T, H) + offs[None, :]).reshape(
        T * H // _STREAM, _STREAM)
    table_cat = jnp.concatenate(tables, axis=0)
    out = _build(T * H)(table_cat, idx_flat)
    return out.reshape(B, S, _NUM_HEADS * _EMBED_DIM)


# speedup vs baseline: 8.5151x; 1.4173x over previous
import functools

import jax
import jax.numpy as jnp
from jax import lax
from jax.experimental import pallas as pl
from jax.experimental.pallas import tpu as pltpu
from jax.experimental.pallas import tpu_sc as plsc

_TABLE_SIZES = (100003, 100019, 100043, 100057, 100069, 100103, 100109,
                100129)
_NUM_HEADS = 8
_EMBED_DIM = 32
_NC = 2
_NS = 16
_NW = _NC * _NS
_CHUNK = 128


@functools.lru_cache(maxsize=None)
def _build(T: int):
    assert T % (_NW * _CHUNK) == 0
    cpw = T // (_NW * _CHUNK)
    assert cpw % 2 == 0
    mesh = plsc.VectorSubcoreMesh(core_axis_name="c", subcore_axis_name="s")

    @functools.partial(
        pl.kernel,
        out_type=jax.ShapeDtypeStruct(
            (T // _CHUNK, _CHUNK, _NUM_HEADS, _EMBED_DIM), jnp.float32),
        mesh=mesh,
        compiler_params=pltpu.CompilerParams(use_tc_tiling_on_sc=False),
        scratch_types=[
            pltpu.VMEM((2, _NUM_HEADS, _CHUNK), jnp.int32),
            pltpu.VMEM((2, _NUM_HEADS, _CHUNK, _EMBED_DIM), jnp.float32),
            pltpu.SemaphoreType.DMA((2,)),
            pltpu.SemaphoreType.DMA((2,)),
        ],
    )
    def gather_kernel(idx_hbm, t0, t1, t2, t3, t4, t5, t6, t7, out_hbm,
                      idx_v, rows_v, gsem, osem):
        tabs = [t0, t1, t2, t3, t4, t5, t6, t7]
        wid = lax.axis_index("s") * _NC + lax.axis_index("c")
        w0 = wid * cpw

        def fire(g, b):
            pltpu.sync_copy(idx_hbm.at[w0 + g], idx_v.at[b])
            for h in range(_NUM_HEADS):
                pltpu.async_copy(tabs[h].at[idx_v.at[b, h]],
                                 rows_v.at[b, h], gsem.at[b])

        def drain_gather(b):
            for h in range(_NUM_HEADS):
                pltpu.make_async_copy(tabs[h].at[idx_v.at[b, h]],
                                      rows_v.at[b, h], gsem.at[b]).wait()

        def write_out(g, b):
            for h in range(_NUM_HEADS):
                pltpu.async_copy(rows_v.at[b, h],
                                 out_hbm.at[w0 + g, :, h, :], osem.at[b])

        def wait_out(g, b):
            for h in range(_NUM_HEADS):
                pltpu.make_async_copy(rows_v.at[b, h],
                                      out_hbm.at[w0 + g, :, h, :],
                                      osem.at[b]).wait()

        fire(0, 0)

        def body(j2, carry):
            for b in range(2):
                g = j2 * 2 + b
                nb = 1 - b

                @pl.when(g + 1 < cpw)
                def _():
                    @pl.when(g >= 1)
                    def _():
                        wait_out(g - 1, nb)
                    fire(g + 1, nb)

                drain_gather(b)
                write_out(g, b)
            return carry

        lax.fori_loop(0, cpw // 2, body, 0)
        wait_out(cpw - 2, 0)
        wait_out(cpw - 1, 1)

    return gather_kernel


def kernel(hash_indices, table_0, table_1, table_2, table_3, table_4,
           table_5, table_6, table_7):
    B, S, H = hash_indices.shape
    T = B * S
    idx3 = hash_indices.reshape(T // _CHUNK, _CHUNK, H).transpose(0, 2, 1)
    out = _build(T)(idx3, table_0, table_1, table_2, table_3, table_4,
                    table_5, table_6, table_7)
    return out.reshape(B, S, _NUM_HEADS * _EMBED_DIM)


# speedup vs baseline: 10.4118x; 1.2227x over previous
import functools

import jax
import jax.numpy as jnp
from jax import lax
from jax.experimental import pallas as pl
from jax.experimental.pallas import tpu as pltpu
from jax.experimental.pallas import tpu_sc as plsc

_TABLE_SIZES = (100003, 100019, 100043, 100057, 100069, 100103, 100109,
                100129)
_NUM_HEADS = 8
_EMBED_DIM = 32
_NC = 2
_NS = 16
_NW = _NC * _NS
_CHUNK = 128


@functools.lru_cache(maxsize=None)
def _build(B: int, S: int):
    assert B == _NW * _CHUNK
    assert S % 2 == 0
    mesh = plsc.VectorSubcoreMesh(core_axis_name="c", subcore_axis_name="s")

    @functools.partial(
        pl.kernel,
        out_type=jax.ShapeDtypeStruct((S, B, _NUM_HEADS * _EMBED_DIM),
                                      jnp.float32),
        mesh=mesh,
        compiler_params=pltpu.CompilerParams(use_tc_tiling_on_sc=False),
        scratch_types=[
            pltpu.VMEM((2, _NUM_HEADS, _CHUNK), jnp.int32),
            pltpu.VMEM((2, _NUM_HEADS, _CHUNK, _EMBED_DIM), jnp.float32),
            pltpu.SemaphoreType.DMA((2,)),
            pltpu.SemaphoreType.DMA((2,)),
        ],
    )
    def gather_kernel(idx_hbm, t0, t1, t2, t3, t4, t5, t6, t7, out_hbm,
                      idx_v, rows_v, gsem, osem):
        tabs = [t0, t1, t2, t3, t4, t5, t6, t7]
        wid = lax.axis_index("s") * _NC + lax.axis_index("c")
        b0 = wid * _CHUNK

        def fire(s, b):
            pltpu.sync_copy(idx_hbm.at[s, :, wid], idx_v.at[b])
            for h in range(_NUM_HEADS):
                pltpu.async_copy(tabs[h].at[idx_v.at[b, h]],
                                 rows_v.at[b, h], gsem.at[b])

        def drain_gather(b):
            for h in range(_NUM_HEADS):
                pltpu.make_async_copy(tabs[h].at[idx_v.at[b, h]],
                                      rows_v.at[b, h], gsem.at[b]).wait()

        def write_out(s, b):
            for h in range(_NUM_HEADS):
                pltpu.async_copy(
                    rows_v.at[b, h],
                    out_hbm.at[s, pl.ds(b0, _CHUNK),
                               pl.ds(h * _EMBED_DIM, _EMBED_DIM)],
                    osem.at[b])

        def wait_out(s, b):
            for h in range(_NUM_HEADS):
                pltpu.make_async_copy(
                    rows_v.at[b, h],
                    out_hbm.at[s, pl.ds(b0, _CHUNK),
                               pl.ds(h * _EMBED_DIM, _EMBED_DIM)],
                    osem.at[b]).wait()

        fire(0, 0)

        def body(j2, carry):
            for b in range(2):
                s = j2 * 2 + b
                nb = 1 - b

                @pl.when(s + 1 < S)
                def _():
                    @pl.when(s >= 1)
                    def _():
                        wait_out(s - 1, nb)
                    fire(s + 1, nb)

                drain_gather(b)
                write_out(s, b)
            return carry

        lax.fori_loop(0, S // 2, body, 0)
        wait_out(S - 2, 0)
        wait_out(S - 1, 1)

    return gather_kernel


def kernel(hash_indices, table_0, table_1, table_2, table_3, table_4,
           table_5, table_6, table_7):
    B, S, H = hash_indices.shape
    idx4 = jnp.transpose(hash_indices, (1, 2, 0)).reshape(
        S, H, B // _CHUNK, _CHUNK)
    out = _build(B, S)(idx4, table_0, table_1, table_2, table_3, table_4,
                       table_5, table_6, table_7)
    return jnp.transpose(out, (1, 0, 2))
